# baseline (speedup 1.0000x reference)
import jax
import jax.numpy as jnp
from jax.experimental import pallas as pl
from jax.experimental.pallas import tpu as pltpu

_K = 17
_PAD = (_K - 1) // 2
_TT = 256


def _batcher_pairs(n):
    pairs = []

    def merge(lo, m, r):
        step = r * 2
        if step < m:
            merge(lo, m, step)
            merge(lo + r, m, step)
            for i in range(lo + r, lo + m - r, step):
                pairs.append((i, i + r))
        else:
            pairs.append((lo, lo + r))

    def sort(lo, m):
        if m > 1:
            h = m // 2
            sort(lo, h)
            sort(lo + h, h)
            merge(lo, m, 1)

    sort(0, n)
    return pairs


def _median17_network():
    n = 32
    state = [(True, False)] * _K + [(False, True)] * (n - _K)
    perm = list(range(n))
    ops = []
    for (i, j) in _batcher_pairs(n):
        ri, ci = state[i]
        rj, cj = state[j]
        if not rj:
            continue
        if not ri:
            perm[i], perm[j] = perm[j], perm[i]
            state[i], state[j] = state[j], state[i]
            continue
        ops.append(("ce", perm[i], perm[j]))
        state[i] = (ri or rj, ci and cj)
        state[j] = (ri or rj, ci or cj)
    out_slot = perm[_K // 2]
    needed = {out_slot}
    kept = []
    for op in reversed(ops):
        _, a, b = op
        if a in needed or b in needed:
            kept.append(op)
            needed.add(a)
            needed.add(b)
    kept.reverse()
    return kept, out_slot


_MEDIAN_OPS, _MEDIAN_OUT = _median17_network()


def _median17(vals):
    vals = list(vals)
    for _, a, b in _MEDIAN_OPS:
        va, vb = vals[a], vals[b]
        vals[a] = jnp.minimum(va, vb)
        vals[b] = jnp.maximum(va, vb)
    return vals[_MEDIAN_OUT]


def _hpss_kernel(x_ref, oh_ref, op_ref, pad_ref):
    t = pl.program_id(1)
    col0 = pl.multiple_of(t * _TT, 128)
    f = oh_ref.shape[2]
    T = x_ref.shape[3]

    @pl.when(t == 0)
    def _():
        pad_ref[0:_PAD, :] = jnp.zeros((_PAD, T + 2 * _PAD), jnp.float32)
        pad_ref[_PAD + f:, :] = jnp.zeros((_PAD, T + 2 * _PAD), jnp.float32)
        pad_ref[:, 0:_PAD] = jnp.zeros((f + 2 * _PAD, _PAD), jnp.float32)
        pad_ref[:, _PAD + T:] = jnp.zeros((f + 2 * _PAD, _PAD), jnp.float32)
        pad_ref[_PAD:_PAD + f, _PAD:_PAD + T] = x_ref[0, 0]

    big = pad_ref[:, pl.ds(col0, _TT + 2 * _PAD)]
    rows = big[_PAD:_PAD + f, :]
    harm = _median17([rows[:, i:i + _TT] for i in range(_K)])
    mid = big[:, _PAD:_PAD + _TT]
    perc = _median17([mid[i:i + f, :] for i in range(_K)])
    s = mid[_PAD:_PAD + f, :]

    z = jnp.maximum(harm, perc)
    tiny = jnp.finfo(jnp.float32).tiny
    z = jnp.where(z < tiny, jnp.float32(1.0), z)
    qh = harm / z
    qp = perc / z
    m = qh * qh
    r = qp * qp
    denom = m + r
    oh_ref[0, 0] = s * (m / denom)
    op_ref[0, 0] = s * (r / denom)


def kernel(S):
    B, C, F, T = S.shape
    nt = T // _TT
    outs = pl.pallas_call(
        _hpss_kernel,
        grid=(B * C, nt),
        in_specs=[
            pl.BlockSpec((1, 1, F, T), lambda b, t: (b // C, b % C, 0, 0))
        ],
        out_specs=[
            pl.BlockSpec((1, 1, F, _TT), lambda b, t: (b // C, b % C, 0, t)),
            pl.BlockSpec((1, 1, F, _TT), lambda b, t: (b // C, b % C, 0, t)),
        ],
        out_shape=[
            jax.ShapeDtypeStruct((B, C, F, T), S.dtype),
            jax.ShapeDtypeStruct((B, C, F, T), S.dtype),
        ],
        scratch_shapes=[
            pltpu.VMEM((F + 2 * _PAD, T + 2 * _PAD), jnp.float32),
        ],
        compiler_params=pltpu.CompilerParams(
            dimension_semantics=("parallel", "arbitrary"),
        ),
        name="hpss_fused",
    )(S)
    return outs[0], outs[1]


# speedup vs baseline: 1.0992x; 1.0992x over previous
import jax
import jax.numpy as jnp
from jax.experimental import pallas as pl
from jax.experimental.pallas import tpu as pltpu

_K = 17
_PAD = (_K - 1) // 2
_TT = 256
_EXT = 2 * _PAD + 15


def _merge_pairs(n):
    pairs = []

    def merge(lo, m, r):
        step = r * 2
        if step < m:
            merge(lo, m, step)
            merge(lo + r, m, step)
            for i in range(lo + r, lo + m - r, step):
                pairs.append((i, i + r))
        else:
            pairs.append((lo, lo + r))

    merge(0, n, 1)
    return pairs


def _pruned_merge16():
    pairs = _merge_pairs(16)
    needed = {7, 8}
    kept = []
    for (a, b) in reversed(pairs):
        if a in needed or b in needed:
            kept.append((a, b))
            needed.add(a)
            needed.add(b)
    kept.reverse()
    return kept


_MERGE4 = _merge_pairs(4)
_MERGE8 = _merge_pairs(8)
_MERGE16_78 = _pruned_merge16()


def _apply(slots, pairs):
    slots = list(slots)
    for i, j in pairs:
        a, b = slots[i], slots[j]
        slots[i] = jnp.minimum(a, b)
        slots[j] = jnp.maximum(a, b)
    return slots


def _sl(a, axis, off, w):
    return a[off:off + w, :] if axis == 0 else a[:, off:off + w]


def _sliding_median17(src, axis, n_out):
    w1 = n_out + _EXT - 1
    s2 = _apply([_sl(src, axis, 0, w1), _sl(src, axis, 1, w1)], [(0, 1)])
    w2 = w1 - 2
    s4 = _apply(
        [_sl(s2[0], axis, 0, w2), _sl(s2[1], axis, 0, w2),
         _sl(s2[0], axis, 2, w2), _sl(s2[1], axis, 2, w2)],
        _MERGE4,
    )
    w3 = w2 - 4
    m8 = _apply(
        [_sl(a, axis, 0, w3) for a in s4] + [_sl(a, axis, 4, w3) for a in s4],
        _MERGE8,
    )
    w4 = w3 - 8
    m16 = _apply(
        [_sl(a, axis, 0, w4) for a in m8] + [_sl(a, axis, 8, w4) for a in m8],
        _MERGE16_78,
    )
    r7 = _sl(m16[7], axis, 0, n_out)
    r8 = _sl(m16[8], axis, 0, n_out)
    x16 = _sl(src, axis, 2 * _PAD, n_out)
    return jnp.minimum(jnp.maximum(x16, r7), r8)


def _hpss_kernel(x_ref, oh_ref, op_ref, pad_ref):
    t = pl.program_id(1)
    col0 = pl.multiple_of(t * _TT, 128)
    f = oh_ref.shape[2]
    T = x_ref.shape[3]
    R = f + _EXT + _PAD
    W = T + _EXT + _PAD

    @pl.when(t == 0)
    def _():
        pad_ref[0:_PAD, :] = jnp.zeros((_PAD, W), jnp.float32)
        pad_ref[_PAD + f:, :] = jnp.zeros((R - _PAD - f, W), jnp.float32)
        pad_ref[:, 0:_PAD] = jnp.zeros((R, _PAD), jnp.float32)
        pad_ref[:, _PAD + T:] = jnp.zeros((R, W - _PAD - T), jnp.float32)
        pad_ref[_PAD:_PAD + f, _PAD:_PAD + T] = x_ref[0, 0]

    big = pad_ref[:, pl.ds(col0, _TT + _EXT)]
    rows = big[_PAD:_PAD + f, :]
    harm = _sliding_median17(rows, 1, _TT)
    mid = big[:, _PAD:_PAD + _TT]
    perc = _sliding_median17(mid, 0, f)
    s = mid[_PAD:_PAD + f, :]

    z = jnp.maximum(harm, perc)
    tiny = jnp.finfo(jnp.float32).tiny
    z = jnp.where(z < tiny, jnp.float32(1.0), z)
    qh = harm / z
    qp = perc / z
    m = qh * qh
    r = qp * qp
    denom = m + r
    oh_ref[0, 0] = s * (m / denom)
    op_ref[0, 0] = s * (r / denom)


def kernel(S):
    B, C, F, T = S.shape
    nt = T // _TT
    outs = pl.pallas_call(
        _hpss_kernel,
        grid=(B * C, nt),
        in_specs=[
            pl.BlockSpec((1, 1, F, T), lambda b, t: (b // C, b % C, 0, 0))
        ],
        out_specs=[
            pl.BlockSpec((1, 1, F, _TT), lambda b, t: (b // C, b % C, 0, t)),
            pl.BlockSpec((1, 1, F, _TT), lambda b, t: (b // C, b % C, 0, t)),
        ],
        out_shape=[
            jax.ShapeDtypeStruct((B, C, F, T), S.dtype),
            jax.ShapeDtypeStruct((B, C, F, T), S.dtype),
        ],
        scratch_shapes=[
            pltpu.VMEM((F + _EXT + _PAD, T + _EXT + _PAD), jnp.float32),
        ],
        compiler_params=pltpu.CompilerParams(
            dimension_semantics=("parallel", "arbitrary"),
        ),
        name="hpss_fused",
    )(S)
    return outs[0], outs[1]


# speedup vs baseline: 1.6440x; 1.4957x over previous
import jax
import jax.numpy as jnp
from jax.experimental import pallas as pl
from jax.experimental.pallas import tpu as pltpu

_K = 17
_PAD = (_K - 1) // 2
_TT = 256
_EXT = 2 * _PAD + 15


def _merge_pairs(n):
    pairs = []

    def merge(lo, m, r):
        step = r * 2
        if step < m:
            merge(lo, m, step)
            merge(lo + r, m, step)
            for i in range(lo + r, lo + m - r, step):
                pairs.append((i, i + r))
        else:
            pairs.append((lo, lo + r))

    merge(0, n, 1)
    return pairs


def _pruned_merge16():
    pairs = _merge_pairs(16)
    needed = {7, 8}
    kept = []
    for (a, b) in reversed(pairs):
        if a in needed or b in needed:
            kept.append((a, b, a in needed, b in needed))
            needed.add(a)
            needed.add(b)
    kept.reverse()
    return kept


_MERGE4 = [(a, b, True, True) for a, b in _merge_pairs(4)]
_MERGE8 = [(a, b, True, True) for a, b in _merge_pairs(8)]
_MERGE16_78 = _pruned_merge16()


def _apply(slots, ops):
    slots = list(slots)
    for a, b, need_min, need_max in ops:
        va, vb = slots[a], slots[b]
        if need_min:
            slots[a] = jnp.minimum(va, vb)
        if need_max:
            slots[b] = jnp.maximum(va, vb)
    return slots


def _sliding_median17(src, n_out):
    w1 = n_out + _EXT - 1
    s2 = _apply([src[0:w1], src[1:1 + w1]], [(0, 1, True, True)])
    w2 = w1 - 2
    s4 = _apply(
        [s2[0][0:w2], s2[1][0:w2], s2[0][2:2 + w2], s2[1][2:2 + w2]],
        _MERGE4,
    )
    w3 = w2 - 4
    m8 = _apply(
        [a[0:w3] for a in s4] + [a[4:4 + w3] for a in s4],
        _MERGE8,
    )
    w4 = w3 - 8
    m16 = _apply(
        [a[0:w4] for a in m8] + [a[8:8 + w4] for a in m8],
        _MERGE16_78,
    )
    r7 = m16[7][0:n_out]
    r8 = m16[8][0:n_out]
    x16 = src[2 * _PAD:2 * _PAD + n_out]
    return jnp.minimum(jnp.maximum(x16, r7), r8)


def _hpss_kernel(x_ref, oh_ref, op_ref, pad_ref, padT_ref):
    t = pl.program_id(1)
    col0 = pl.multiple_of(t * _TT, 128)
    f = oh_ref.shape[2]
    T = x_ref.shape[3]
    R = pad_ref.shape[0]
    RT = padT_ref.shape[0]

    @pl.when(t == 0)
    def _():
        pad_ref[0:_PAD, :] = jnp.zeros((_PAD, T), jnp.float32)
        pad_ref[_PAD + f:, :] = jnp.zeros((R - _PAD - f, T), jnp.float32)
        pad_ref[_PAD:_PAD + f, :] = x_ref[0, 0]
        padT_ref[0:_PAD, :] = jnp.zeros((_PAD, f), jnp.float32)
        padT_ref[_PAD + T:, :] = jnp.zeros((RT - _PAD - T, f), jnp.float32)
        for j in range(T // _TT):
            padT_ref[_PAD + j * _TT:_PAD + (j + 1) * _TT, :] = jnp.transpose(
                x_ref[0, 0, :, j * _TT:(j + 1) * _TT]
            )

    srcT = padT_ref[pl.ds(col0, _TT + _EXT), :]
    harm = jnp.transpose(_sliding_median17(srcT, _TT))

    srcF = pad_ref[:, pl.ds(col0, _TT)]
    perc = _sliding_median17(srcF, f)
    s = srcF[_PAD:_PAD + f, :]

    z = jnp.maximum(harm, perc)
    tiny = jnp.finfo(jnp.float32).tiny
    z = jnp.where(z < tiny, jnp.float32(1.0), z)
    qh = harm / z
    qp = perc / z
    m = qh * qh
    r = qp * qp
    denom = m + r
    oh_ref[0, 0] = s * (m / denom)
    op_ref[0, 0] = s * (r / denom)


def kernel(S):
    B, C, F, T = S.shape
    nt = T // _TT
    rows_f = F + _EXT + _PAD
    rows_t = -(-(T + _EXT + _PAD) // 8) * 8
    outs = pl.pallas_call(
        _hpss_kernel,
        grid=(B * C, nt),
        in_specs=[
            pl.BlockSpec((1, 1, F, T), lambda b, t: (b // C, b % C, 0, 0))
        ],
        out_specs=[
            pl.BlockSpec((1, 1, F, _TT), lambda b, t: (b // C, b % C, 0, t)),
            pl.BlockSpec((1, 1, F, _TT), lambda b, t: (b // C, b % C, 0, t)),
        ],
        out_shape=[
            jax.ShapeDtypeStruct((B, C, F, T), S.dtype),
            jax.ShapeDtypeStruct((B, C, F, T), S.dtype),
        ],
        scratch_shapes=[
            pltpu.VMEM((rows_f, T), jnp.float32),
            pltpu.VMEM((rows_t, F), jnp.float32),
        ],
        compiler_params=pltpu.CompilerParams(
            dimension_semantics=("parallel", "arbitrary"),
            vmem_limit_bytes=56 * 1024 * 1024,
        ),
        name="hpss_fused",
    )(S)
    return outs[0], outs[1]


# speedup vs baseline: 2.1324x; 1.2970x over previous
import jax
import jax.numpy as jnp
from jax.experimental import pallas as pl
from jax.experimental.pallas import tpu as pltpu

_K = 17
_PAD = (_K - 1) // 2
_TT = 256
_EXT = 2 * _PAD + 15


def _merge_pairs(n):
    pairs = []

    def merge(lo, m, r):
        step = r * 2
        if step < m:
            merge(lo, m, step)
            merge(lo + r, m, step)
            for i in range(lo + r, lo + m - r, step):
                pairs.append((i, i + r))
        else:
            pairs.append((lo, lo + r))

    merge(0, n, 1)
    return pairs


def _pruned_merge16():
    pairs = _merge_pairs(16)
    needed = {7, 8}
    kept = []
    for (a, b) in reversed(pairs):
        if a in needed or b in needed:
            kept.append((a, b, a in needed, b in needed))
            needed.add(a)
            needed.add(b)
    kept.reverse()
    return kept


_MERGE4 = [(a, b, True, True) for a, b in _merge_pairs(4)]
_MERGE8 = [(a, b, True, True) for a, b in _merge_pairs(8)]
_MERGE16_78 = _pruned_merge16()


def _apply(slots, ops):
    slots = list(slots)
    for a, b, need_min, need_max in ops:
        va, vb = slots[a], slots[b]
        if need_min:
            slots[a] = jnp.minimum(va, vb)
        if need_max:
            slots[b] = jnp.maximum(va, vb)
    return slots


def _sliding_median17(src, n_out):
    w1 = n_out + _EXT - 1
    s2 = _apply([src[0:w1], src[1:1 + w1]], [(0, 1, True, True)])
    w2 = w1 - 2
    s4 = _apply(
        [s2[0][0:w2], s2[1][0:w2], s2[0][2:2 + w2], s2[1][2:2 + w2]],
        _MERGE4,
    )
    w3 = w2 - 4
    m8 = _apply(
        [a[0:w3] for a in s4] + [a[4:4 + w3] for a in s4],
        _MERGE8,
    )
    w4 = w3 - 8
    m16 = _apply(
        [a[0:w4] for a in m8] + [a[8:8 + w4] for a in m8],
        _MERGE16_78,
    )
    r7 = m16[7][0:n_out]
    r8 = m16[8][0:n_out]
    x16 = src[2 * _PAD:2 * _PAD + n_out]
    return jnp.minimum(jnp.maximum(x16, r7), r8)


def _hpss_kernel(x_ref, oh_ref, op_ref, pad_ref, padT_ref):
    t = pl.program_id(1)
    col0 = pl.multiple_of(t * _TT, 128)
    f = oh_ref.shape[2]
    T = x_ref.shape[3]
    R = pad_ref.shape[0]
    RT = padT_ref.shape[0]

    @pl.when(t == 0)
    def _():
        xb = x_ref[0, 0].astype(jnp.bfloat16)
        pad_ref[0:_PAD, :] = jnp.zeros((_PAD, T), jnp.bfloat16)
        pad_ref[_PAD + f:, :] = jnp.zeros((R - _PAD - f, T), jnp.bfloat16)
        pad_ref[_PAD:_PAD + f, :] = xb
        padT_ref[0:_PAD, :] = jnp.zeros((_PAD, f), jnp.bfloat16)
        padT_ref[_PAD + T:, :] = jnp.zeros((RT - _PAD - T, f), jnp.bfloat16)
        for j in range(T // _TT):
            padT_ref[_PAD + j * _TT:_PAD + (j + 1) * _TT, :] = jnp.transpose(
                xb[:, j * _TT:(j + 1) * _TT]
            )

    srcT = padT_ref[pl.ds(col0, _TT + _EXT), :]
    harm = jnp.transpose(_sliding_median17(srcT, _TT)).astype(jnp.float32)

    srcF = pad_ref[:, pl.ds(col0, _TT)]
    perc = _sliding_median17(srcF, f).astype(jnp.float32)
    s = x_ref[0, 0, :, pl.ds(col0, _TT)]

    z = jnp.maximum(harm, perc)
    tiny = jnp.finfo(jnp.float32).tiny
    z = jnp.where(z < tiny, jnp.float32(1.0), z)
    qh = harm / z
    qp = perc / z
    m = qh * qh
    r = qp * qp
    denom = m + r
    oh_ref[0, 0] = s * (m / denom)
    op_ref[0, 0] = s * (r / denom)


def kernel(S):
    B, C, F, T = S.shape
    nt = T // _TT
    rows_f = F + _EXT + _PAD
    rows_t = -(-(T + _EXT + _PAD) // 8) * 8
    outs = pl.pallas_call(
        _hpss_kernel,
        grid=(B * C, nt),
        in_specs=[
            pl.BlockSpec((1, 1, F, T), lambda b, t: (b // C, b % C, 0, 0))
        ],
        out_specs=[
            pl.BlockSpec((1, 1, F, _TT), lambda b, t: (b // C, b % C, 0, t)),
            pl.BlockSpec((1, 1, F, _TT), lambda b, t: (b // C, b % C, 0, t)),
        ],
        out_shape=[
            jax.ShapeDtypeStruct((B, C, F, T), S.dtype),
            jax.ShapeDtypeStruct((B, C, F, T), S.dtype),
        ],
        scratch_shapes=[
            pltpu.VMEM((rows_f, T), jnp.bfloat16),
            pltpu.VMEM((rows_t, F), jnp.bfloat16),
        ],
        compiler_params=pltpu.CompilerParams(
            dimension_semantics=("parallel", "arbitrary"),
            vmem_limit_bytes=56 * 1024 * 1024,
        ),
        name="hpss_fused",
    )(S)
    return outs[0], outs[1]
